# baseline (speedup 1.0000x reference)
import functools
import math

import jax
import jax.numpy as jnp
from jax import lax
from jax.experimental import pallas as pl
from jax.experimental.pallas import tpu as pltpu
from jax.experimental.pallas import tpu_sc as plsc

_BOUNDS = tuple(float(b) for b in range(1, 100, 10))
_INV_SQRT = 1.0 / math.sqrt(1.0 + 1e-3)
_ITEM_PAD = 128
_N_BUCKETS = 11

_T_CONTRACT = (((0,), (0,)), ((), ()))
_T_CONTRACT_R1 = (((0,), (1,)), ((), ()))


def _prologue_body(price_ref, item_ref, pt_ref, it_ref, g1_ref, be1_ref,
                   w1_ref, b1_ref, g2_ref, be2_ref, w2_ref, b2_ref,
                   combo_ref, table_ref):
    p = price_ref[...]
    bucket = jnp.zeros(p.shape, jnp.int32)
    for bound in _BOUNDS:
        bucket = bucket + jnp.where(p >= bound, 1, 0)
    combo_ref[...] = bucket * _ITEM_PAD + item_ref[...]

    n_items = it_ref.shape[0]
    w1s = w1_ref[...] * (g1_ref[...] * _INV_SQRT)
    at = lax.dot_general(w1s[:32, :], pt_ref[...], _T_CONTRACT_R1,
                         precision=lax.Precision.HIGHEST,
                         preferred_element_type=jnp.float32)
    ct = lax.dot_general(w1s[32:, :], it_ref[...], _T_CONTRACT_R1,
                         precision=lax.Precision.HIGHEST,
                         preferred_element_type=jnp.float32)
    ct = jnp.concatenate(
        [ct, jnp.zeros((32, _ITEM_PAD - n_items), jnp.float32)], axis=1)
    d1t = lax.dot_general(w1_ref[...], be1_ref[...], _T_CONTRACT,
                          precision=lax.Precision.HIGHEST,
                          preferred_element_type=jnp.float32) + b1_ref[...]
    w2s = w2_ref[...] * (g2_ref[...] * _INV_SQRT)
    d2t = lax.dot_general(w2_ref[...], be2_ref[...], _T_CONTRACT,
                          precision=lax.Precision.HIGHEST,
                          preferred_element_type=jnp.float32) + b2_ref[...]
    for b in range(_N_BUCKETS):
        ht = jnp.maximum(at[:, b:b + 1] + ct + d1t, 0.0)
        tt = jnp.maximum(
            lax.dot_general(w2s, ht, _T_CONTRACT,
                            precision=lax.Precision.HIGHEST,
                            preferred_element_type=jnp.float32) + d2t,
            0.0)
        table_ref[:, b * _ITEM_PAD:(b + 1) * _ITEM_PAD] = tt


def _prologue(price2, item2, pt, it, g1c, be1c, w1, b1c, g2c, be2c, w2, b2c):
    return pl.pallas_call(
        _prologue_body,
        out_shape=(
            jax.ShapeDtypeStruct(price2.shape, jnp.int32),
            jax.ShapeDtypeStruct((16, _N_BUCKETS * _ITEM_PAD), jnp.float32),
        ),
    )(price2, item2, pt, it, g1c, be1c, w1, b1c, g2c, be2c, w2, b2c)


@functools.cache
def _make_sc_lookup(batch):
    info = plsc.get_sparse_core_info()
    n_workers = info.num_cores * info.num_subcores
    bpw = batch // n_workers
    n_combos = _N_BUCKETS * _ITEM_PAD
    mesh = plsc.VectorSubcoreMesh(core_axis_name="c", subcore_axis_name="s")

    @functools.partial(
        pl.kernel,
        mesh=mesh,
        out_type=jax.ShapeDtypeStruct((batch, 16), jnp.float32),
        compiler_params=pltpu.CompilerParams(needs_layout_passes=False),
        scratch_types=[
            pltpu.VMEM((bpw,), jnp.int32),
            pltpu.VMEM((16, n_combos), jnp.float32),
            pltpu.VMEM((bpw, 16), jnp.float32),
            pltpu.SemaphoreType.DMA,
            pltpu.SemaphoreType.DMA,
        ],
    )
    def sc_lookup(combo_hbm, table_hbm, out_hbm,
                  combo_v, table_v, rows_v, sem, tsem):
        wid = lax.axis_index("s") * info.num_cores + lax.axis_index("c")
        base = wid * bpw
        tcopy = pltpu.async_copy(table_hbm, table_v, tsem)
        ccopy = pltpu.async_copy(combo_hbm.at[pl.ds(base, bpw)], combo_v, sem)
        ccopy.wait()
        tcopy.wait()
        iota = lax.iota(jnp.int32, 16)
        for g in range(bpw // 16):
            combo = combo_v[pl.ds(g * 16, 16)]
            rid = iota + g * 16
            for col in range(16):
                colv = jnp.full((16,), col, jnp.int32)
                vals = plsc.load_gather(table_v, [colv, combo])
                plsc.store_scatter(rows_v, [rid, colv], vals)
        pltpu.sync_copy(rows_v, out_hbm.at[pl.ds(base, bpw)])

    return sc_lookup


def kernel(user_id, item_id, price, user_age, item_table, price_table,
           bn1_gamma, bn1_beta, W1, b1, bn2_gamma, bn2_beta, W2, b2):
    batch = price.shape[0]
    combo2, table = _prologue(
        price.reshape(batch // 128, 128), item_id.reshape(batch // 128, 128),
        price_table, item_table, bn1_gamma.reshape(64, 1) ,
        bn1_beta.reshape(64, 1), W1, b1.reshape(32, 1),
        bn2_gamma.reshape(32, 1), bn2_beta.reshape(32, 1),
        W2, b2.reshape(16, 1))
    return _make_sc_lookup(batch)(combo2.reshape(batch), table)


# speedup vs baseline: 1.1771x; 1.1771x over previous
import functools
import math

import jax
import jax.numpy as jnp
from jax import lax
from jax.experimental import pallas as pl
from jax.experimental.pallas import tpu as pltpu
from jax.experimental.pallas import tpu_sc as plsc

_BOUNDS = tuple(float(b) for b in range(1, 100, 10))
_INV_SQRT = 1.0 / math.sqrt(1.0 + 1e-3)
_ITEM_PAD = 128
_N_BUCKETS = 11

_T_CONTRACT = (((0,), (0,)), ((), ()))
_T_CONTRACT_R1 = (((0,), (1,)), ((), ()))
_HI = lax.Precision.HIGHEST


def _tdot(a, b, dims):
    return lax.dot_general(a, b, dims, precision=_HI,
                           preferred_element_type=jnp.float32)


def _prologue_body(price_ref, item_ref, pt_ref, it_ref, g1_ref, be1_ref,
                   w1_ref, b1_ref, g2_ref, be2_ref, w2_ref, b2_ref,
                   combo_ref, table_ref):
    p = price_ref[...]
    bucket = jnp.zeros(p.shape, jnp.int32)
    for bound in _BOUNDS:
        bucket = bucket + jnp.where(p >= bound, 1, 0)
    combo_ref[...] = bucket * _ITEM_PAD + item_ref[...]

    n_items = it_ref.shape[0]
    w1 = w1_ref[...]
    g1 = g1_ref[...] * _INV_SQRT
    be1 = be1_ref[...]
    w1t, w1b = w1[:32, :], w1[32:, :]
    ptg = pt_ref[...] * g1[:, :32]
    d1a = jnp.dot(be1[:, :32], w1t, precision=_HI,
                  preferred_element_type=jnp.float32) + b1_ref[...]
    ptaug = jnp.concatenate(
        [ptg, jnp.ones((ptg.shape[0], 1), jnp.float32)], axis=1)
    w1taug = jnp.concatenate([w1t, d1a], axis=0)
    at = _tdot(w1taug, ptaug, _T_CONTRACT_R1)
    itg = it_ref[...] * g1[:, 32:]
    d1b = jnp.dot(be1[:, 32:], w1b, precision=_HI,
                  preferred_element_type=jnp.float32)
    itaug = jnp.concatenate(
        [itg, jnp.ones((n_items, 1), jnp.float32)], axis=1)
    w1baug = jnp.concatenate([w1b, d1b], axis=0)
    ct = _tdot(w1baug, itaug, _T_CONTRACT_R1)
    ct = jnp.concatenate(
        [ct, jnp.zeros((32, _ITEM_PAD - n_items), jnp.float32)], axis=1)

    w2 = w2_ref[...]
    g2 = g2_ref[...] * _INV_SQRT
    r32 = lax.broadcasted_iota(jnp.int32, (32, 32), 0)
    c32 = lax.broadcasted_iota(jnp.int32, (32, 32), 1)
    dg2 = jnp.where(r32 == c32, 1.0, 0.0) * g2
    w2s = jnp.dot(dg2, w2, precision=_HI,
                  preferred_element_type=jnp.float32)
    d2 = jnp.dot(be2_ref[...], w2, precision=_HI,
                 preferred_element_type=jnp.float32) + b2_ref[...]
    w2aug = jnp.concatenate([w2s, d2], axis=0)
    ones_row = jnp.ones((1, _ITEM_PAD), jnp.float32)
    for b in range(_N_BUCKETS):
        ht = jnp.maximum(at[:, b:b + 1] + ct, 0.0)
        htaug = jnp.concatenate([ht, ones_row], axis=0)
        tt = jnp.maximum(_tdot(w2aug, htaug, _T_CONTRACT), 0.0)
        table_ref[:, b * _ITEM_PAD:(b + 1) * _ITEM_PAD] = tt


def _prologue(price, item, pt, it, g1r, be1r, w1, b1r, g2r, be2r, w2, b2r):
    return pl.pallas_call(
        _prologue_body,
        out_shape=(
            jax.ShapeDtypeStruct(price.shape, jnp.int32),
            jax.ShapeDtypeStruct((16, _N_BUCKETS * _ITEM_PAD), jnp.float32),
        ),
    )(price, item, pt, it, g1r, be1r, w1, b1r, g2r, be2r, w2, b2r)


@functools.cache
def _make_sc_lookup(batch):
    info = plsc.get_sparse_core_info()
    n_workers = info.num_cores * info.num_subcores
    bpw = batch // n_workers
    n_combos = _N_BUCKETS * _ITEM_PAD
    mesh = plsc.VectorSubcoreMesh(core_axis_name="c", subcore_axis_name="s")

    @functools.partial(
        pl.kernel,
        mesh=mesh,
        out_type=jax.ShapeDtypeStruct((batch, 16), jnp.float32),
        compiler_params=pltpu.CompilerParams(needs_layout_passes=False),
        scratch_types=[
            pltpu.VMEM((bpw,), jnp.int32),
            pltpu.VMEM((16, n_combos), jnp.float32),
            pltpu.VMEM((bpw, 16), jnp.float32),
            pltpu.SemaphoreType.DMA,
            pltpu.SemaphoreType.DMA,
        ],
    )
    def sc_lookup(combo_hbm, table_hbm, out_hbm,
                  combo_v, table_v, rows_v, sem, tsem):
        wid = lax.axis_index("s") * info.num_cores + lax.axis_index("c")
        base = wid * bpw
        tcopy = pltpu.async_copy(table_hbm, table_v, tsem)
        ccopy = pltpu.async_copy(combo_hbm.at[pl.ds(base, bpw)], combo_v, sem)
        ccopy.wait()
        tcopy.wait()
        iota = lax.iota(jnp.int32, 16)
        for g in range(bpw // 16):
            combo = combo_v[pl.ds(g * 16, 16)]
            rid = iota + g * 16
            for col in range(16):
                colv = jnp.full((16,), col, jnp.int32)
                vals = plsc.load_gather(table_v, [colv, combo])
                plsc.store_scatter(rows_v, [rid, colv], vals)
        pltpu.sync_copy(rows_v, out_hbm.at[pl.ds(base, bpw)])

    return sc_lookup


def kernel(user_id, item_id, price, user_age, item_table, price_table,
           bn1_gamma, bn1_beta, W1, b1, bn2_gamma, bn2_beta, W2, b2):
    combo, table = _prologue(
        price, item_id, price_table, item_table,
        bn1_gamma.reshape(1, 64), bn1_beta.reshape(1, 64), W1,
        b1.reshape(1, 32), bn2_gamma.reshape(1, 32), bn2_beta.reshape(1, 32),
        W2, b2.reshape(1, 16))
    return _make_sc_lookup(price.shape[0])(combo, table)


# speedup vs baseline: 1.6864x; 1.4326x over previous
import functools
import math

import jax
import jax.numpy as jnp
from jax import lax
from jax.experimental import pallas as pl
from jax.experimental.pallas import tpu as pltpu
from jax.experimental.pallas import tpu_sc as plsc

_BOUNDS = tuple(float(b) for b in range(1, 100, 10))
_INV_SQRT = 1.0 / math.sqrt(1.0 + 1e-3)
_ITEM_PAD = 128
_N_BUCKETS = 11

_T_CONTRACT = (((0,), (0,)), ((), ()))
_T_CONTRACT_R1 = (((0,), (1,)), ((), ()))
_HI = lax.Precision.HIGHEST


def _tdot(a, b, dims):
    return lax.dot_general(a, b, dims, precision=_HI,
                           preferred_element_type=jnp.float32)


def _prologue_body(price_ref, item_ref, pt_ref, it_ref, g1_ref, be1_ref,
                   w1_ref, b1_ref, g2_ref, be2_ref, w2_ref, b2_ref,
                   combo_ref, table_ref):
    p = price_ref[...]
    bucket = jnp.zeros(p.shape, jnp.int32)
    for bound in _BOUNDS:
        bucket = bucket + jnp.where(p >= bound, 1, 0)
    combo_ref[...] = bucket * _ITEM_PAD + item_ref[...]

    n_items = it_ref.shape[0]
    w1 = w1_ref[...]
    g1 = g1_ref[...] * _INV_SQRT
    be1 = be1_ref[...]
    w1t, w1b = w1[:32, :], w1[32:, :]
    ptg = pt_ref[...] * g1[:, :32]
    d1a = jnp.dot(be1[:, :32], w1t, precision=_HI,
                  preferred_element_type=jnp.float32) + b1_ref[...]
    ptaug = jnp.concatenate(
        [ptg, jnp.ones((ptg.shape[0], 1), jnp.float32)], axis=1)
    w1taug = jnp.concatenate([w1t, d1a], axis=0)
    at = _tdot(w1taug, ptaug, _T_CONTRACT_R1)
    itg = it_ref[...] * g1[:, 32:]
    d1b = jnp.dot(be1[:, 32:], w1b, precision=_HI,
                  preferred_element_type=jnp.float32)
    itaug = jnp.concatenate(
        [itg, jnp.ones((n_items, 1), jnp.float32)], axis=1)
    w1baug = jnp.concatenate([w1b, d1b], axis=0)
    ct = _tdot(w1baug, itaug, _T_CONTRACT_R1)
    ct = jnp.concatenate(
        [ct, jnp.zeros((32, _ITEM_PAD - n_items), jnp.float32)], axis=1)

    w2 = w2_ref[...]
    g2 = g2_ref[...] * _INV_SQRT
    r32 = lax.broadcasted_iota(jnp.int32, (32, 32), 0)
    c32 = lax.broadcasted_iota(jnp.int32, (32, 32), 1)
    dg2 = jnp.where(r32 == c32, 1.0, 0.0) * g2
    w2s = jnp.dot(dg2, w2, precision=_HI,
                  preferred_element_type=jnp.float32)
    d2 = jnp.dot(be2_ref[...], w2, precision=_HI,
                 preferred_element_type=jnp.float32) + b2_ref[...]
    w2aug = jnp.concatenate([w2s, d2], axis=0)
    ones_row = jnp.ones((1, _ITEM_PAD), jnp.float32)
    for b in range(_N_BUCKETS):
        ht = jnp.maximum(at[:, b:b + 1] + ct, 0.0)
        htaug = jnp.concatenate([ht, ones_row], axis=0)
        tt = jnp.maximum(_tdot(w2aug, htaug, _T_CONTRACT), 0.0)
        table_ref[:, b * _ITEM_PAD:(b + 1) * _ITEM_PAD] = tt


def _prologue(price, item, pt, it, g1r, be1r, w1, b1r, g2r, be2r, w2, b2r):
    return pl.pallas_call(
        _prologue_body,
        out_shape=(
            jax.ShapeDtypeStruct(price.shape, jnp.int32),
            jax.ShapeDtypeStruct((16, _N_BUCKETS * _ITEM_PAD), jnp.float32),
        ),
    )(price, item, pt, it, g1r, be1r, w1, b1r, g2r, be2r, w2, b2r)


@functools.cache
def _make_sc_lookup(batch):
    info = plsc.get_sparse_core_info()
    n_workers = info.num_cores * info.num_subcores
    bpw = batch // n_workers
    n_combos = _N_BUCKETS * _ITEM_PAD
    mesh = plsc.VectorSubcoreMesh(core_axis_name="c", subcore_axis_name="s")

    @functools.partial(
        pl.kernel,
        mesh=mesh,
        out_type=jax.ShapeDtypeStruct((16, batch), jnp.float32),
        compiler_params=pltpu.CompilerParams(needs_layout_passes=False),
        scratch_types=[
            pltpu.VMEM((bpw,), jnp.int32),
            pltpu.VMEM((16, n_combos), jnp.float32),
            pltpu.VMEM((16, bpw), jnp.float32),
            pltpu.SemaphoreType.DMA,
            pltpu.SemaphoreType.DMA,
        ],
    )
    def sc_lookup(combo_hbm, table_hbm, out_hbm,
                  combo_v, table_v, rows_t, sem, tsem):
        wid = lax.axis_index("s") * info.num_cores + lax.axis_index("c")
        base = wid * bpw
        tcopy = pltpu.async_copy(table_hbm, table_v, tsem)
        ccopy = pltpu.async_copy(combo_hbm.at[pl.ds(base, bpw)], combo_v, sem)
        ccopy.wait()
        tcopy.wait()
        for g in range(bpw // 16):
            combo = combo_v[pl.ds(g * 16, 16)]
            for col in range(16):
                colv = jnp.full((16,), col, jnp.int32)
                vals = plsc.load_gather(table_v, [colv, combo])
                rows_t[col, pl.ds(g * 16, 16)] = vals
        pltpu.sync_copy(rows_t, out_hbm.at[:, pl.ds(base, bpw)])

    return sc_lookup


def kernel(user_id, item_id, price, user_age, item_table, price_table,
           bn1_gamma, bn1_beta, W1, b1, bn2_gamma, bn2_beta, W2, b2):
    combo, table = _prologue(
        price, item_id, price_table, item_table,
        bn1_gamma.reshape(1, 64), bn1_beta.reshape(1, 64), W1,
        b1.reshape(1, 32), bn2_gamma.reshape(1, 32), bn2_beta.reshape(1, 32),
        W2, b2.reshape(1, 16))
    out_t = _make_sc_lookup(price.shape[0])(combo, table)
    return out_t.T


# speedup vs baseline: 1.6879x; 1.0009x over previous
import functools
import math

import jax
import jax.numpy as jnp
from jax import lax
from jax.experimental import pallas as pl
from jax.experimental.pallas import tpu as pltpu
from jax.experimental.pallas import tpu_sc as plsc

_BOUNDS = tuple(float(b) for b in range(1, 100, 10))
_INV_SQRT = 1.0 / math.sqrt(1.0 + 1e-3)
_ITEM_PAD = 128
_N_BUCKETS = 11

_T_CONTRACT = (((0,), (0,)), ((), ()))
_T_CONTRACT_R1 = (((0,), (1,)), ((), ()))
_HI = lax.Precision.HIGHEST


def _tdot(a, b, dims):
    return lax.dot_general(a, b, dims, precision=_HI,
                           preferred_element_type=jnp.float32)


def _prologue_body(price_hbm, item_hbm, pt_hbm, it_hbm, g1_hbm, be1_hbm,
                   w1_hbm, b1_hbm, g2_hbm, be2_hbm, w2_hbm, b2_hbm,
                   combo_ref, table_ref,
                   price_ref, item_ref, pt_ref, it_ref, g1_ref, be1_ref,
                   w1_ref, b1_ref, g2_ref, be2_ref, w2_ref, b2_ref, sem):
    pairs = [(price_hbm, price_ref), (item_hbm, item_ref), (pt_hbm, pt_ref),
             (it_hbm, it_ref), (g1_hbm, g1_ref), (be1_hbm, be1_ref),
             (w1_hbm, w1_ref), (b1_hbm, b1_ref), (g2_hbm, g2_ref),
             (be2_hbm, be2_ref), (w2_hbm, w2_ref), (b2_hbm, b2_ref)]
    copies = [pltpu.make_async_copy(s, d, sem) for s, d in pairs]
    for c in copies:
        c.start()
    for c in copies:
        c.wait()

    p = price_ref[...]
    bucket = jnp.zeros(p.shape, jnp.int32)
    for bound in _BOUNDS:
        bucket = bucket + jnp.where(p >= bound, 1, 0)
    combo_ref[...] = bucket * _ITEM_PAD + item_ref[...]

    n_items = it_ref.shape[0]
    w1 = w1_ref[...]
    g1 = g1_ref[...] * _INV_SQRT
    be1 = be1_ref[...]
    w1t, w1b = w1[:32, :], w1[32:, :]
    ptg = pt_ref[...] * g1[:, :32]
    d1a = jnp.dot(be1[:, :32], w1t, precision=_HI,
                  preferred_element_type=jnp.float32) + b1_ref[...]
    ptaug = jnp.concatenate(
        [ptg, jnp.ones((ptg.shape[0], 1), jnp.float32)], axis=1)
    w1taug = jnp.concatenate([w1t, d1a], axis=0)
    at = _tdot(w1taug, ptaug, _T_CONTRACT_R1)
    itg = it_ref[...] * g1[:, 32:]
    d1b = jnp.dot(be1[:, 32:], w1b, precision=_HI,
                  preferred_element_type=jnp.float32)
    itaug = jnp.concatenate(
        [itg, jnp.ones((n_items, 1), jnp.float32)], axis=1)
    w1baug = jnp.concatenate([w1b, d1b], axis=0)
    ct = _tdot(w1baug, itaug, _T_CONTRACT_R1)
    ct = jnp.concatenate(
        [ct, jnp.zeros((32, _ITEM_PAD - n_items), jnp.float32)], axis=1)

    w2 = w2_ref[...]
    g2 = g2_ref[...] * _INV_SQRT
    r32 = lax.broadcasted_iota(jnp.int32, (32, 32), 0)
    c32 = lax.broadcasted_iota(jnp.int32, (32, 32), 1)
    dg2 = jnp.where(r32 == c32, 1.0, 0.0) * g2
    w2s = jnp.dot(dg2, w2, precision=_HI,
                  preferred_element_type=jnp.float32)
    d2 = jnp.dot(be2_ref[...], w2, precision=_HI,
                 preferred_element_type=jnp.float32) + b2_ref[...]
    w2aug = jnp.concatenate([w2s, d2], axis=0)
    n_all = _N_BUCKETS * _ITEM_PAD
    ht = jnp.maximum(at[:, :, None] + ct[:, None, :], 0.0)
    htaug = jnp.concatenate(
        [ht.reshape(32, n_all), jnp.ones((1, n_all), jnp.float32)], axis=0)
    table_ref[...] = jnp.maximum(_tdot(w2aug, htaug, _T_CONTRACT), 0.0)


def _prologue(price, item, pt, it, g1r, be1r, w1, b1r, g2r, be2r, w2, b2r):
    any_spec = pl.BlockSpec(memory_space=pl.ANY)
    return pl.pallas_call(
        _prologue_body,
        in_specs=[any_spec] * 12,
        out_shape=(
            jax.ShapeDtypeStruct(price.shape, jnp.int32),
            jax.ShapeDtypeStruct((16, _N_BUCKETS * _ITEM_PAD), jnp.float32),
        ),
        scratch_shapes=[
            pltpu.VMEM(price.shape, jnp.float32),
            pltpu.VMEM(item.shape, jnp.int32),
            pltpu.VMEM(pt.shape, jnp.float32),
            pltpu.VMEM(it.shape, jnp.float32),
            pltpu.VMEM(g1r.shape, jnp.float32),
            pltpu.VMEM(be1r.shape, jnp.float32),
            pltpu.VMEM(w1.shape, jnp.float32),
            pltpu.VMEM(b1r.shape, jnp.float32),
            pltpu.VMEM(g2r.shape, jnp.float32),
            pltpu.VMEM(be2r.shape, jnp.float32),
            pltpu.VMEM(w2.shape, jnp.float32),
            pltpu.VMEM(b2r.shape, jnp.float32),
            pltpu.SemaphoreType.DMA,
        ],
    )(price, item, pt, it, g1r, be1r, w1, b1r, g2r, be2r, w2, b2r)


@functools.cache
def _make_sc_lookup(batch):
    info = plsc.get_sparse_core_info()
    n_workers = info.num_cores * info.num_subcores
    bpw = batch // n_workers
    n_combos = _N_BUCKETS * _ITEM_PAD
    mesh = plsc.VectorSubcoreMesh(core_axis_name="c", subcore_axis_name="s")

    @functools.partial(
        pl.kernel,
        mesh=mesh,
        out_type=jax.ShapeDtypeStruct((16, batch), jnp.float32),
        compiler_params=pltpu.CompilerParams(needs_layout_passes=False),
        scratch_types=[
            pltpu.VMEM((bpw,), jnp.int32),
            pltpu.VMEM((16, n_combos), jnp.float32),
            pltpu.VMEM((16, bpw), jnp.float32),
            pltpu.SemaphoreType.DMA,
            pltpu.SemaphoreType.DMA,
        ],
    )
    def sc_lookup(combo_hbm, table_hbm, out_hbm,
                  combo_v, table_v, rows_t, sem, tsem):
        wid = lax.axis_index("s") * info.num_cores + lax.axis_index("c")
        base = wid * bpw
        tcopy = pltpu.async_copy(table_hbm, table_v, tsem)
        ccopy = pltpu.async_copy(combo_hbm.at[pl.ds(base, bpw)], combo_v, sem)
        ccopy.wait()
        tcopy.wait()
        for g in range(bpw // 16):
            combo = combo_v[pl.ds(g * 16, 16)]
            for col in range(16):
                colv = jnp.full((16,), col, jnp.int32)
                vals = plsc.load_gather(table_v, [colv, combo])
                rows_t[col, pl.ds(g * 16, 16)] = vals
        pltpu.sync_copy(rows_t, out_hbm.at[:, pl.ds(base, bpw)])

    return sc_lookup


def kernel(user_id, item_id, price, user_age, item_table, price_table,
           bn1_gamma, bn1_beta, W1, b1, bn2_gamma, bn2_beta, W2, b2):
    combo, table = _prologue(
        price, item_id, price_table, item_table,
        bn1_gamma.reshape(1, 64), bn1_beta.reshape(1, 64), W1,
        b1.reshape(1, 32), bn2_gamma.reshape(1, 32), bn2_beta.reshape(1, 32),
        W2, b2.reshape(1, 16))
    out_t = _make_sc_lookup(price.shape[0])(combo, table)
    return out_t.T


# speedup vs baseline: 1.7734x; 1.0506x over previous
import functools
import math

import jax
import jax.numpy as jnp
from jax import lax
from jax.experimental import pallas as pl
from jax.experimental.pallas import tpu as pltpu
from jax.experimental.pallas import tpu_sc as plsc

_BOUNDS = tuple(float(b) for b in range(1, 100, 10))
_INV_SQRT = 1.0 / math.sqrt(1.0 + 1e-3)
_ITEM_PAD = 128
_N_BUCKETS = 11

_T_CONTRACT = (((0,), (0,)), ((), ()))
_T_CONTRACT_R1 = (((0,), (1,)), ((), ()))
_HI = lax.Precision.HIGHEST


def _tdot(a, b, dims):
    return lax.dot_general(a, b, dims, precision=_HI,
                           preferred_element_type=jnp.float32)


def _prologue_body(price_hbm, item_hbm, pt_hbm, it_hbm, g1_hbm, be1_hbm,
                   w1_hbm, b1_hbm, g2_hbm, be2_hbm, w2_hbm, b2_hbm,
                   combo_ref, table_ref,
                   price_ref, item_ref, pt_ref, it_ref, g1_ref, be1_ref,
                   w1_ref, b1_ref, g2_ref, be2_ref, w2_ref, b2_ref, sem):
    pairs = [(price_hbm, price_ref), (item_hbm, item_ref), (pt_hbm, pt_ref),
             (it_hbm, it_ref), (g1_hbm, g1_ref), (be1_hbm, be1_ref),
             (w1_hbm, w1_ref), (b1_hbm, b1_ref), (g2_hbm, g2_ref),
             (be2_hbm, be2_ref), (w2_hbm, w2_ref), (b2_hbm, b2_ref)]
    copies = [pltpu.make_async_copy(s, d, sem) for s, d in pairs]
    for c in copies:
        c.start()
    for c in copies:
        c.wait()

    p = price_ref[...]
    bucket = jnp.zeros(p.shape, jnp.int32)
    for bound in _BOUNDS:
        bucket = bucket + jnp.where(p >= bound, 1, 0)
    combo_ref[...] = bucket * _ITEM_PAD + item_ref[...]

    n_items = it_ref.shape[0]
    w1 = w1_ref[...]
    g1 = g1_ref[...] * _INV_SQRT
    be1 = be1_ref[...]
    w1t, w1b = w1[:32, :], w1[32:, :]
    ptg = pt_ref[...] * g1[:, :32]
    d1a = jnp.dot(be1[:, :32], w1t, precision=_HI,
                  preferred_element_type=jnp.float32) + b1_ref[...]
    ptaug = jnp.concatenate(
        [ptg, jnp.ones((ptg.shape[0], 1), jnp.float32)], axis=1)
    w1taug = jnp.concatenate([w1t, d1a], axis=0)
    at = _tdot(w1taug, ptaug, _T_CONTRACT_R1)
    itg = it_ref[...] * g1[:, 32:]
    d1b = jnp.dot(be1[:, 32:], w1b, precision=_HI,
                  preferred_element_type=jnp.float32)
    itaug = jnp.concatenate(
        [itg, jnp.ones((n_items, 1), jnp.float32)], axis=1)
    w1baug = jnp.concatenate([w1b, d1b], axis=0)
    ct = _tdot(w1baug, itaug, _T_CONTRACT_R1)
    ct = jnp.concatenate(
        [ct, jnp.zeros((32, _ITEM_PAD - n_items), jnp.float32)], axis=1)

    w2 = w2_ref[...]
    g2 = g2_ref[...] * _INV_SQRT
    r32 = lax.broadcasted_iota(jnp.int32, (32, 32), 0)
    c32 = lax.broadcasted_iota(jnp.int32, (32, 32), 1)
    dg2 = jnp.where(r32 == c32, 1.0, 0.0) * g2
    w2s = jnp.dot(dg2, w2, precision=_HI,
                  preferred_element_type=jnp.float32)
    d2 = jnp.dot(be2_ref[...], w2, precision=_HI,
                 preferred_element_type=jnp.float32) + b2_ref[...]
    w2aug = jnp.concatenate([w2s, d2], axis=0)
    n_all = _N_BUCKETS * _ITEM_PAD
    ht = jnp.maximum(at[:, :, None] + ct[:, None, :], 0.0)
    htaug = jnp.concatenate(
        [ht.reshape(32, n_all), jnp.ones((1, n_all), jnp.float32)], axis=0)
    table_ref[...] = jnp.maximum(_tdot(w2aug, htaug, _T_CONTRACT), 0.0)


def _prologue(price, item, pt, it, g1r, be1r, w1, b1r, g2r, be2r, w2, b2r):
    any_spec = pl.BlockSpec(memory_space=pl.ANY)
    return pl.pallas_call(
        _prologue_body,
        in_specs=[any_spec] * 12,
        out_shape=(
            jax.ShapeDtypeStruct(price.shape, jnp.int32),
            jax.ShapeDtypeStruct((16, _N_BUCKETS * _ITEM_PAD), jnp.float32),
        ),
        scratch_shapes=[
            pltpu.VMEM(price.shape, jnp.float32),
            pltpu.VMEM(item.shape, jnp.int32),
            pltpu.VMEM(pt.shape, jnp.float32),
            pltpu.VMEM(it.shape, jnp.float32),
            pltpu.VMEM(g1r.shape, jnp.float32),
            pltpu.VMEM(be1r.shape, jnp.float32),
            pltpu.VMEM(w1.shape, jnp.float32),
            pltpu.VMEM(b1r.shape, jnp.float32),
            pltpu.VMEM(g2r.shape, jnp.float32),
            pltpu.VMEM(be2r.shape, jnp.float32),
            pltpu.VMEM(w2.shape, jnp.float32),
            pltpu.VMEM(b2r.shape, jnp.float32),
            pltpu.SemaphoreType.DMA,
        ],
    )(price, item, pt, it, g1r, be1r, w1, b1r, g2r, be2r, w2, b2r)


@functools.cache
def _make_sc_lookup(batch):
    info = plsc.get_sparse_core_info()
    n_workers = info.num_cores * info.num_subcores
    bpw = batch // n_workers
    n_combos = _N_BUCKETS * _ITEM_PAD
    mesh = plsc.VectorSubcoreMesh(core_axis_name="c", subcore_axis_name="s")

    @functools.partial(
        pl.kernel,
        mesh=mesh,
        out_type=jax.ShapeDtypeStruct((16, batch), jnp.float32),
        compiler_params=pltpu.CompilerParams(needs_layout_passes=False),
        scratch_types=[
            pltpu.VMEM((bpw,), jnp.int32),
            pltpu.VMEM((16, n_combos), jnp.float32),
            pltpu.VMEM((16, bpw), jnp.float32),
            pltpu.SemaphoreType.DMA,
            pltpu.SemaphoreType.DMA,
        ],
    )
    def sc_lookup(combo_hbm, table_hbm, out_hbm,
                  combo_v, table_v, rows_t, sem, tsem):
        wid = lax.axis_index("s") * info.num_cores + lax.axis_index("c")
        base = wid * bpw
        tcopy = pltpu.async_copy(table_hbm, table_v, tsem)
        ccopy = pltpu.async_copy(combo_hbm.at[pl.ds(base, bpw)], combo_v, sem)
        ccopy.wait()
        tcopy.wait()
        def gather_group(g, carry):
            base16 = g * 16
            combo = combo_v[pl.ds(base16, 16)]
            for col in range(16):
                colv = jnp.full((16,), col, jnp.int32)
                vals = plsc.load_gather(table_v, [colv, combo])
                rows_t[col, pl.ds(base16, 16)] = vals
            return carry

        lax.fori_loop(0, bpw // 16, gather_group, 0)
        pltpu.sync_copy(rows_t, out_hbm.at[:, pl.ds(base, bpw)])

    return sc_lookup


def kernel(user_id, item_id, price, user_age, item_table, price_table,
           bn1_gamma, bn1_beta, W1, b1, bn2_gamma, bn2_beta, W2, b2):
    combo, table = _prologue(
        price, item_id, price_table, item_table,
        bn1_gamma.reshape(1, 64), bn1_beta.reshape(1, 64), W1,
        b1.reshape(1, 32), bn2_gamma.reshape(1, 32), bn2_beta.reshape(1, 32),
        W2, b2.reshape(1, 16))
    out_t = _make_sc_lookup(price.shape[0])(combo, table)
    return out_t.T


# speedup vs baseline: 1.9234x; 1.0846x over previous
import functools
import math

import jax
import jax.numpy as jnp
from jax import lax
from jax.experimental import pallas as pl
from jax.experimental.pallas import tpu as pltpu
from jax.experimental.pallas import tpu_sc as plsc

_BOUNDS = tuple(float(b) for b in range(1, 100, 10))
_INV_SQRT = 1.0 / math.sqrt(1.0 + 1e-3)
_ITEM_PAD = 128
_N_BUCKETS = 11

_T_CONTRACT = (((0,), (0,)), ((), ()))
_T_CONTRACT_R1 = (((0,), (1,)), ((), ()))
_HI = lax.Precision.HIGHEST


def _tdot(a, b, dims):
    return lax.dot_general(a, b, dims, precision=_HI,
                           preferred_element_type=jnp.float32)


def _prologue_body(price_hbm, item_hbm, pt_hbm, it_hbm, g1_hbm, be1_hbm,
                   w1_hbm, b1_hbm, g2_hbm, be2_hbm, w2_hbm, b2_hbm,
                   combo_ref, table_ref,
                   price_ref, item_ref, pt_ref, it_ref, g1_ref, be1_ref,
                   w1_ref, b1_ref, g2_ref, be2_ref, w2_ref, b2_ref, sem):
    pairs = [(price_hbm, price_ref), (item_hbm, item_ref), (pt_hbm, pt_ref),
             (it_hbm, it_ref), (g1_hbm, g1_ref), (be1_hbm, be1_ref),
             (w1_hbm, w1_ref), (b1_hbm, b1_ref), (g2_hbm, g2_ref),
             (be2_hbm, be2_ref), (w2_hbm, w2_ref), (b2_hbm, b2_ref)]
    copies = [pltpu.make_async_copy(s, d, sem) for s, d in pairs]
    for c in copies:
        c.start()
    for c in copies:
        c.wait()

    p = price_ref[...]
    bucket = jnp.zeros(p.shape, jnp.int32)
    for bound in _BOUNDS:
        bucket = bucket + jnp.where(p >= bound, 1, 0)
    combo_ref[...] = bucket * _ITEM_PAD + item_ref[...]

    n_items = it_ref.shape[0]
    w1 = w1_ref[...]
    g1 = g1_ref[...] * _INV_SQRT
    be1 = be1_ref[...]
    w1t, w1b = w1[:32, :], w1[32:, :]
    ptg = pt_ref[...] * g1[:, :32]
    d1a = jnp.dot(be1[:, :32], w1t, precision=_HI,
                  preferred_element_type=jnp.float32) + b1_ref[...]
    ptaug = jnp.concatenate(
        [ptg, jnp.ones((ptg.shape[0], 1), jnp.float32)], axis=1)
    w1taug = jnp.concatenate([w1t, d1a], axis=0)
    at = _tdot(w1taug, ptaug, _T_CONTRACT_R1)
    itg = it_ref[...] * g1[:, 32:]
    d1b = jnp.dot(be1[:, 32:], w1b, precision=_HI,
                  preferred_element_type=jnp.float32)
    itaug = jnp.concatenate(
        [itg, jnp.ones((n_items, 1), jnp.float32)], axis=1)
    w1baug = jnp.concatenate([w1b, d1b], axis=0)
    ct = _tdot(w1baug, itaug, _T_CONTRACT_R1)
    ct = jnp.concatenate(
        [ct, jnp.zeros((32, _ITEM_PAD - n_items), jnp.float32)], axis=1)

    w2 = w2_ref[...]
    g2 = g2_ref[...] * _INV_SQRT
    r32 = lax.broadcasted_iota(jnp.int32, (32, 32), 0)
    c32 = lax.broadcasted_iota(jnp.int32, (32, 32), 1)
    dg2 = jnp.where(r32 == c32, 1.0, 0.0) * g2
    w2s = jnp.dot(dg2, w2, precision=_HI,
                  preferred_element_type=jnp.float32)
    d2 = jnp.dot(be2_ref[...], w2, precision=_HI,
                 preferred_element_type=jnp.float32) + b2_ref[...]
    w2aug = jnp.concatenate([w2s, d2], axis=0)
    n_all = _N_BUCKETS * _ITEM_PAD
    ht = jnp.maximum(at[:, :, None] + ct[:, None, :], 0.0)
    htaug = jnp.concatenate(
        [ht.reshape(32, n_all), jnp.ones((1, n_all), jnp.float32)], axis=0)
    table_ref[...] = jnp.maximum(_tdot(w2aug, htaug, _T_CONTRACT), 0.0)


def _prologue(price, item, pt, it, g1r, be1r, w1, b1r, g2r, be2r, w2, b2r):
    any_spec = pl.BlockSpec(memory_space=pl.ANY)
    return pl.pallas_call(
        _prologue_body,
        in_specs=[any_spec] * 12,
        out_shape=(
            jax.ShapeDtypeStruct(price.shape, jnp.int32),
            jax.ShapeDtypeStruct((16, _N_BUCKETS * _ITEM_PAD), jnp.float32),
        ),
        scratch_shapes=[
            pltpu.VMEM(price.shape, jnp.float32),
            pltpu.VMEM(item.shape, jnp.int32),
            pltpu.VMEM(pt.shape, jnp.float32),
            pltpu.VMEM(it.shape, jnp.float32),
            pltpu.VMEM(g1r.shape, jnp.float32),
            pltpu.VMEM(be1r.shape, jnp.float32),
            pltpu.VMEM(w1.shape, jnp.float32),
            pltpu.VMEM(b1r.shape, jnp.float32),
            pltpu.VMEM(g2r.shape, jnp.float32),
            pltpu.VMEM(be2r.shape, jnp.float32),
            pltpu.VMEM(w2.shape, jnp.float32),
            pltpu.VMEM(b2r.shape, jnp.float32),
            pltpu.SemaphoreType.DMA,
        ],
    )(price, item, pt, it, g1r, be1r, w1, b1r, g2r, be2r, w2, b2r)


@functools.cache
def _make_sc_lookup(batch):
    info = plsc.get_sparse_core_info()
    n_workers = info.num_cores * info.num_subcores
    bpw = batch // n_workers
    n_combos = _N_BUCKETS * _ITEM_PAD
    mesh = plsc.VectorSubcoreMesh(core_axis_name="c", subcore_axis_name="s")

    half = batch // 2

    @functools.partial(
        pl.kernel,
        mesh=mesh,
        out_type=jax.ShapeDtypeStruct((16, batch), jnp.float32),
        compiler_params=pltpu.CompilerParams(needs_layout_passes=False),
        scratch_types=[
            pltpu.VMEM((half,), jnp.int32),
            pltpu.VMEM((1, n_combos), jnp.float32),
            pltpu.VMEM((1, half), jnp.float32),
            pltpu.SemaphoreType.DMA,
            pltpu.SemaphoreType.DMA,
        ],
    )
    def sc_lookup(combo_hbm, table_hbm, out_hbm,
                  combo_v, trow_v, rows_v, sem, tsem):
        wid = lax.axis_index("s") * info.num_cores + lax.axis_index("c")
        col = lax.rem(wid, 16)
        base = lax.div(wid, 16) * half
        tcopy = pltpu.async_copy(table_hbm.at[pl.ds(col, 1), :], trow_v, tsem)
        ccopy = pltpu.async_copy(combo_hbm.at[pl.ds(base, half)], combo_v, sem)
        ccopy.wait()
        tcopy.wait()
        zero16 = jnp.full((16,), 0, jnp.int32)

        def gather_group(g, carry):
            for j in range(8):
                b16 = g * 128 + j * 16
                combo = combo_v[pl.ds(b16, 16)]
                vals = plsc.load_gather(trow_v, [zero16, combo])
                rows_v[0, pl.ds(b16, 16)] = vals
            return carry

        lax.fori_loop(0, half // 128, gather_group, 0)
        pltpu.sync_copy(rows_v, out_hbm.at[pl.ds(col, 1), pl.ds(base, half)])

    return sc_lookup


def kernel(user_id, item_id, price, user_age, item_table, price_table,
           bn1_gamma, bn1_beta, W1, b1, bn2_gamma, bn2_beta, W2, b2):
    combo, table = _prologue(
        price, item_id, price_table, item_table,
        bn1_gamma.reshape(1, 64), bn1_beta.reshape(1, 64), W1,
        b1.reshape(1, 32), bn2_gamma.reshape(1, 32), bn2_beta.reshape(1, 32),
        W2, b2.reshape(1, 16))
    out_t = _make_sc_lookup(price.shape[0])(combo, table)
    return out_t.T
